# baseline (speedup 1.0000x reference)
import jax
import jax.numpy as jnp
from jax import lax
from jax.experimental import pallas as pl
from jax.experimental.pallas import tpu as pltpu
from jax.experimental.pallas import tpu_sc as plsc

NC = 2
NS = 16
NW = NC * NS
LANES = 16

CHUNK_B = 32
GW = 100


def _masked_segment_sum(emb_weight, feats_flat, bid_flat, B, L, D):
    rows_per_tile = B // NW
    chunk_n = CHUNK_B * L
    ng = chunk_n // GW
    n_chunks = rows_per_tile // CHUNK_B
    n_iota = chunk_n // LANES

    mesh = plsc.VectorSubcoreMesh(core_axis_name="c", subcore_axis_name="s")

    def body(table, feats_f, bid_f, out, idx_v, bid_v,
             rows_v, zbuf, comp, sg0, sg1, ss0, ss1):
        c = lax.axis_index("c")
        s = lax.axis_index("s")
        wid = c * NS + s
        gr0 = wid * rows_per_tile
        lr0 = s * rows_per_tile

        z16 = jnp.zeros((LANES,), jnp.float32)
        for r in range(LANES):
            zbuf[r, pl.ds(0, LANES)] = z16
            zbuf[r, pl.ds(LANES, LANES)] = z16
        for j in range(rows_per_tile // LANES):
            pltpu.sync_copy(zbuf, comp.at[pl.ds(lr0 + j * LANES, LANES)])

        sgs = (sg0, sg1)
        sss = (ss0, ss1)
        rows_per_chunk2d = chunk_n // GW

        def load_and_gather(cc, p):
            roff = (gr0 * L + cc * chunk_n) // GW
            pltpu.sync_copy(feats_f.at[pl.ds(roff, ng)], idx_v.at[p])
            pltpu.sync_copy(bid_f.at[pl.ds(roff, ng)], bid_v.at[p])
            for j in range(ng):
                pltpu.async_copy(table.at[idx_v.at[p, j]],
                                 rows_v.at[p, pl.ds(j * GW, GW)], sgs[p])

        def wait_gathers(p):
            for j in range(ng):
                pltpu.make_async_copy(table.at[idx_v.at[p, j]],
                                      rows_v.at[p, pl.ds(j * GW, GW)],
                                      sgs[p]).wait()

        def scatter_add(p):
            for j in range(ng):
                pltpu.async_copy(rows_v.at[p, pl.ds(j * GW, GW)],
                                 comp.at[bid_v.at[p, j]], sss[p], add=True)

        def drain_scatters(p):
            for j in range(ng):
                pltpu.make_async_copy(rows_v.at[p, pl.ds(j * GW, GW)],
                                      comp.at[bid_v.at[p, j]],
                                      sss[p]).wait()

        load_and_gather(0, 0)
        load_and_gather(1, 1)

        def pair(k, carry):
            wait_gathers(0)
            scatter_add(0)
            wait_gathers(1)
            scatter_add(1)
            drain_scatters(0)
            load_and_gather(2 * k + 2, 0)
            drain_scatters(1)
            load_and_gather(2 * k + 3, 1)
            return carry

        lax.fori_loop(0, n_chunks // 2 - 1, pair, 0)
        wait_gathers(0)
        scatter_add(0)
        wait_gathers(1)
        scatter_add(1)
        drain_scatters(0)
        drain_scatters(1)

        pltpu.sync_copy(comp.at[pl.ds(lr0, rows_per_tile)],
                        out.at[pl.ds(gr0, rows_per_tile)])

    return pl.kernel(
        body,
        out_type=jax.ShapeDtypeStruct((B, D), jnp.float32),
        mesh=mesh,
        compiler_params=pltpu.CompilerParams(use_tc_tiling_on_sc=False),
        scratch_types=[
            pltpu.VMEM((2, ng, GW), jnp.int32),
            pltpu.VMEM((2, ng, GW), jnp.int32),
            pltpu.VMEM((2, chunk_n, D), jnp.float32),
            pltpu.VMEM((LANES, D), jnp.float32),
            pltpu.VMEM_SHARED((B // NC + NS, D), jnp.float32),
            pltpu.SemaphoreType.DMA,
            pltpu.SemaphoreType.DMA,
            pltpu.SemaphoreType.DMA,
            pltpu.SemaphoreType.DMA,
        ],
    )(emb_weight, feats_flat, bid_flat)


def _detile_table(embT):
    D, V = embT.shape
    CB = 8192
    grid = ((V + CB - 1) // CB,)

    def body(in_ref, out_ref):
        eye = jnp.eye(D, dtype=jnp.float32)
        out_ref[...] = lax.dot_general(
            in_ref[...], eye, (((0,), (0,)), ((), ())),
            preferred_element_type=jnp.float32,
            precision=lax.Precision.HIGHEST)

    return pl.pallas_call(
        body,
        grid=grid,
        in_specs=[pl.BlockSpec((D, CB), lambda k: (0, k))],
        out_specs=pl.BlockSpec((CB, D), lambda k: (k, 0)),
        out_shape=jax.ShapeDtypeStruct((V, D), jnp.float32),
    )(embT)


def _cosine_epilogue(S, rep, maskf):
    Bn = S.shape[0]

    def body(s_ref, rep_ref, m_ref, o_ref):
        sv = s_ref[...]
        r = rep_ref[...]
        m = m_ref[...]
        denom = jnp.maximum(jnp.sum(m, axis=1, keepdims=True), 1e-6)
        comp = sv / denom
        cn = jnp.maximum(jnp.sqrt(jnp.sum(comp * comp, axis=1, keepdims=True)),
                         1e-8)
        rn = jnp.maximum(jnp.sqrt(jnp.sum(r * r, axis=1, keepdims=True)),
                         1e-8)
        cos = jnp.sum(comp * r, axis=1, keepdims=True) / (cn * rn)
        o_ref[...] = 1.0 - cos

    BB = 2048
    return pl.pallas_call(
        body,
        grid=(Bn // BB,),
        in_specs=[
            pl.BlockSpec((BB, S.shape[1]), lambda i: (i, 0)),
            pl.BlockSpec((BB, rep.shape[1]), lambda i: (i, 0)),
            pl.BlockSpec((BB, maskf.shape[1]), lambda i: (i, 0)),
        ],
        out_specs=pl.BlockSpec((BB, 1), lambda i: (i, 0)),
        out_shape=jax.ShapeDtypeStruct((Bn, 1), jnp.float32),
    )(S, rep, maskf)


def kernel(rep, feats, feats_mask, emb_weight):
    B, L = feats.shape
    D = emb_weight.shape[1]
    rows_per_tile = B // NW
    feats_flat = feats.astype(jnp.int32).reshape(B * L // GW, GW)
    brow = lax.broadcasted_iota(jnp.int32, (B, L), 0)
    local = brow % (B // NC)
    trash = (B // NC) + (brow // rows_per_tile) % NS
    bid = jnp.where(feats_mask, local, trash)
    bid_flat = bid.astype(jnp.int32).reshape(B * L // GW, GW)
    table_lin = _detile_table(emb_weight.T)
    S = _masked_segment_sum(table_lin, feats_flat, bid_flat, B, L, D)
    maskf = feats_mask.astype(jnp.float32)
    out = _cosine_epilogue(S, rep, maskf)
    return out.reshape(B)


# speedup vs baseline: 1.8038x; 1.8038x over previous
import jax
import jax.numpy as jnp
from jax import lax
from jax.experimental import pallas as pl
from jax.experimental.pallas import tpu as pltpu
from jax.experimental.pallas import tpu_sc as plsc

NC = 2
NS = 16
NW = NC * NS
LANES = 16

CHUNK_B = 32
GW = 100


def _masked_segment_sum(emb_weight, feats_flat, bid_flat, B, L, D):
    rows_per_tile = B // NW
    chunk_n = CHUNK_B * L
    ng = chunk_n // GW
    n_chunks = rows_per_tile // CHUNK_B
    n_iota = chunk_n // LANES

    mesh = plsc.VectorSubcoreMesh(core_axis_name="c", subcore_axis_name="s")

    def body(table, feats_f, bid_f, out, idx_v, bid_v,
             rows_v, zbuf, comp, sg0, sg1, ss0, ss1):
        c = lax.axis_index("c")
        s = lax.axis_index("s")
        wid = c * NS + s
        gr0 = wid * rows_per_tile
        lr0 = s * rows_per_tile

        z16 = jnp.zeros((LANES,), jnp.float32)
        for r in range(LANES):
            zbuf[r, pl.ds(0, LANES)] = z16
            zbuf[r, pl.ds(LANES, LANES)] = z16
        for j in range(rows_per_tile // LANES):
            pltpu.sync_copy(zbuf, comp.at[pl.ds(lr0 + j * LANES, LANES)])

        sgs = (sg0, sg1)
        sss = (ss0, ss1)
        rows_per_chunk2d = chunk_n // GW

        def load_and_gather(cc, p):
            roff = (gr0 * L + cc * chunk_n) // GW
            pltpu.sync_copy(feats_f.at[pl.ds(roff, ng)], idx_v.at[p])
            pltpu.sync_copy(bid_f.at[pl.ds(roff, ng)], bid_v.at[p])
            for j in range(ng):
                pltpu.async_copy(table.at[idx_v.at[p, j]],
                                 rows_v.at[p, pl.ds(j * GW, GW)], sgs[p])

        def wait_gathers(p):
            for j in range(ng):
                pltpu.make_async_copy(table.at[idx_v.at[p, j]],
                                      rows_v.at[p, pl.ds(j * GW, GW)],
                                      sgs[p]).wait()

        def scatter_add(p):
            for j in range(ng):
                pltpu.async_copy(rows_v.at[p, pl.ds(j * GW, GW)],
                                 comp.at[bid_v.at[p, j]], sss[p], add=True)

        def drain_scatters(p):
            for j in range(ng):
                pltpu.make_async_copy(rows_v.at[p, pl.ds(j * GW, GW)],
                                      comp.at[bid_v.at[p, j]],
                                      sss[p]).wait()

        load_and_gather(0, 0)
        load_and_gather(1, 1)

        def pair(k, carry):
            wait_gathers(0)
            scatter_add(0)
            wait_gathers(1)
            scatter_add(1)
            drain_scatters(0)
            load_and_gather(2 * k + 2, 0)
            drain_scatters(1)
            load_and_gather(2 * k + 3, 1)
            return carry

        lax.fori_loop(0, n_chunks // 2 - 1, pair, 0)
        wait_gathers(0)
        scatter_add(0)
        wait_gathers(1)
        scatter_add(1)
        drain_scatters(0)
        drain_scatters(1)

        pltpu.sync_copy(comp.at[pl.ds(lr0, rows_per_tile)],
                        out.at[pl.ds(gr0, rows_per_tile)])

    return pl.kernel(
        body,
        out_type=jax.ShapeDtypeStruct((B, D), jnp.float32),
        mesh=mesh,
        compiler_params=pltpu.CompilerParams(use_tc_tiling_on_sc=False),
        scratch_types=[
            pltpu.VMEM((2, ng, GW), jnp.int32),
            pltpu.VMEM((2, ng, GW), jnp.int32),
            pltpu.VMEM((2, chunk_n, D), jnp.float32),
            pltpu.VMEM((LANES, D), jnp.float32),
            pltpu.VMEM_SHARED((B // NC + NS, D), jnp.float32),
            pltpu.SemaphoreType.DMA,
            pltpu.SemaphoreType.DMA,
            pltpu.SemaphoreType.DMA,
            pltpu.SemaphoreType.DMA,
        ],
    )(emb_weight, feats_flat, bid_flat)


def _detile_table(embT):
    D, V = embT.shape
    CB = 8192
    grid = ((V + CB - 1) // CB,)

    def body(in_ref, out_ref):
        y = in_ref[...].T.reshape(CB // 4, 4, D)
        for q in range(4):
            out_ref[:, q * D:(q + 1) * D] = y[:, q, :]

    return pl.pallas_call(
        body,
        grid=grid,
        in_specs=[pl.BlockSpec((D, CB), lambda k: (0, k))],
        out_specs=pl.BlockSpec((CB // 4, 4 * D), lambda k: (k, 0)),
        out_shape=jax.ShapeDtypeStruct((V // 4, 4 * D), jnp.float32),
    )(embT)


def _cosine_epilogue(S, rep, maskf):
    Bn = S.shape[0]

    def body(s_ref, rep_ref, m_ref, o_ref):
        sv = s_ref[...]
        r = rep_ref[...]
        m = m_ref[...]
        denom = jnp.maximum(jnp.sum(m, axis=1, keepdims=True), 1e-6)
        comp = sv / denom
        cn = jnp.maximum(jnp.sqrt(jnp.sum(comp * comp, axis=1, keepdims=True)),
                         1e-8)
        rn = jnp.maximum(jnp.sqrt(jnp.sum(r * r, axis=1, keepdims=True)),
                         1e-8)
        cos = jnp.sum(comp * r, axis=1, keepdims=True) / (cn * rn)
        o_ref[...] = 1.0 - cos

    BB = 2048
    return pl.pallas_call(
        body,
        grid=(Bn // BB,),
        in_specs=[
            pl.BlockSpec((BB, S.shape[1]), lambda i: (i, 0)),
            pl.BlockSpec((BB, rep.shape[1]), lambda i: (i, 0)),
            pl.BlockSpec((BB, maskf.shape[1]), lambda i: (i, 0)),
        ],
        out_specs=pl.BlockSpec((BB, 1), lambda i: (i, 0)),
        out_shape=jax.ShapeDtypeStruct((Bn, 1), jnp.float32),
    )(S, rep, maskf)


def kernel(rep, feats, feats_mask, emb_weight):
    B, L = feats.shape
    D = emb_weight.shape[1]
    rows_per_tile = B // NW
    feats_flat = feats.astype(jnp.int32).reshape(B * L // GW, GW)
    brow = lax.broadcasted_iota(jnp.int32, (B, L), 0)
    local = brow % (B // NC)
    trash = (B // NC) + (brow // rows_per_tile) % NS
    bid = jnp.where(feats_mask, local, trash)
    bid_flat = bid.astype(jnp.int32).reshape(B * L // GW, GW)
    table_lin = _detile_table(emb_weight.T).reshape(emb_weight.shape)
    S = _masked_segment_sum(table_lin, feats_flat, bid_flat, B, L, D)
    maskf = feats_mask.astype(jnp.float32)
    out = _cosine_epilogue(S, rep, maskf)
    return out.reshape(B)
